# Initial kernel scaffold; baseline (speedup 1.0000x reference)
import jax
import jax.numpy as jnp
from jax.experimental import pallas as pl


def kernel(x, weight, bias):
    raise NotImplementedError("write your pallas kernel here")


# speedup vs baseline: 1.7100x; 1.7100x over previous
import jax
import jax.numpy as jnp
from jax.experimental import pallas as pl
from jax.experimental.pallas import tpu as pltpu


def _conv_row_kernel(x_ref, w_ref, b_ref, o_ref):
    acc = jnp.dot(w_ref[...], x_ref[...], preferred_element_type=jnp.float32)
    o_ref[...] = (acc + b_ref[...]).astype(o_ref.dtype)


def kernel(x, weight, bias):
    N, C_in, L = x.shape
    C_out = weight.shape[0]

    w2d = weight[:, :, 0]
    b2d = bias.reshape(C_out, 1)

    grid = (N,)
    return pl.pallas_call(
        _conv_row_kernel,
        out_shape=jax.ShapeDtypeStruct((N, C_out, L), x.dtype),
        grid=grid,
        in_specs=[
            pl.BlockSpec((None, C_in, L), lambda n: (n, 0, 0)),
            pl.BlockSpec((C_out, C_in), lambda n: (0, 0)),
            pl.BlockSpec((C_out, 1), lambda n: (0, 0)),
        ],
        out_specs=pl.BlockSpec((None, C_out, L), lambda n: (n, 0, 0)),
        compiler_params=pltpu.CompilerParams(
            dimension_semantics=("parallel",),
            vmem_limit_bytes=40 * 1024 * 1024,
        ),
    )(x, w2d, b2d)


# speedup vs baseline: 1.7169x; 1.0040x over previous
import jax
import jax.numpy as jnp
from jax.experimental import pallas as pl
from jax.experimental.pallas import tpu as pltpu


def _conv_row_kernel(xa_ref, xb_ref, w_ref, b_ref, o_ref):
    h = xa_ref.shape[1]
    acc_a = jnp.dot(w_ref[...], xa_ref[...], preferred_element_type=jnp.float32)
    o_ref[:, :h] = (acc_a + b_ref[...]).astype(o_ref.dtype)
    acc_b = jnp.dot(w_ref[...], xb_ref[...], preferred_element_type=jnp.float32)
    o_ref[:, h:] = (acc_b + b_ref[...]).astype(o_ref.dtype)


def kernel(x, weight, bias):
    N, C_in, L = x.shape
    C_out = weight.shape[0]
    H = L // 2

    w2d = weight[:, :, 0]
    b2d = bias.reshape(C_out, 1)

    grid = (N,)
    return pl.pallas_call(
        _conv_row_kernel,
        out_shape=jax.ShapeDtypeStruct((N, C_out, L), x.dtype),
        grid=grid,
        in_specs=[
            pl.BlockSpec((None, C_in, H), lambda n: (n, 0, 0)),
            pl.BlockSpec((None, C_in, H), lambda n: (n, 0, 1)),
            pl.BlockSpec((C_out, C_in), lambda n: (0, 0)),
            pl.BlockSpec((C_out, 1), lambda n: (0, 0)),
        ],
        out_specs=pl.BlockSpec((None, C_out, L), lambda n: (n, 0, 0)),
        compiler_params=pltpu.CompilerParams(
            dimension_semantics=("parallel",),
            vmem_limit_bytes=48 * 1024 * 1024,
        ),
    )(x, x, w2d, b2d)


# speedup vs baseline: 1.7716x; 1.0319x over previous
import jax
import jax.numpy as jnp
from jax.experimental import pallas as pl
from jax.experimental.pallas import tpu as pltpu


def _conv_rows_kernel(x_ref, w_ref, b_ref, o_ref):
    for i in range(x_ref.shape[0]):
        acc = jnp.dot(w_ref[...], x_ref[i], preferred_element_type=jnp.float32)
        o_ref[i] = (acc + b_ref[...]).astype(o_ref.dtype)


def kernel(x, weight, bias):
    N, C_in, L = x.shape
    C_out = weight.shape[0]
    R = 2

    w2d = weight[:, :, 0]
    b2d = bias.reshape(C_out, 1)

    grid = (N // R,)
    return pl.pallas_call(
        _conv_rows_kernel,
        out_shape=jax.ShapeDtypeStruct((N, C_out, L), x.dtype),
        grid=grid,
        in_specs=[
            pl.BlockSpec((R, C_in, L), lambda n: (n, 0, 0)),
            pl.BlockSpec((C_out, C_in), lambda n: (0, 0)),
            pl.BlockSpec((C_out, 1), lambda n: (0, 0)),
        ],
        out_specs=pl.BlockSpec((R, C_out, L), lambda n: (n, 0, 0)),
        compiler_params=pltpu.CompilerParams(
            dimension_semantics=("parallel",),
            vmem_limit_bytes=40 * 1024 * 1024,
        ),
    )(x, w2d, b2d)


# speedup vs baseline: 1.7744x; 1.0016x over previous
import jax
import jax.numpy as jnp
from jax.experimental import pallas as pl
from jax.experimental.pallas import tpu as pltpu


def _conv_rows_kernel(x_ref, w_ref, b_ref, o_ref):
    for i in range(x_ref.shape[0]):
        acc = jnp.dot(w_ref[...], x_ref[i], preferred_element_type=jnp.float32)
        o_ref[i] = (acc + b_ref[...]).astype(o_ref.dtype)


def kernel(x, weight, bias):
    N, C_in, L = x.shape
    C_out = weight.shape[0]
    R = 2 if N % 2 == 0 else 1

    w2d = weight[:, :, 0]
    b2d = bias.reshape(C_out, 1)

    grid = (N // R,)
    return pl.pallas_call(
        _conv_rows_kernel,
        out_shape=jax.ShapeDtypeStruct((N, C_out, L), x.dtype),
        grid=grid,
        in_specs=[
            pl.BlockSpec((R, C_in, L), lambda n: (n, 0, 0)),
            pl.BlockSpec((C_out, C_in), lambda n: (0, 0)),
            pl.BlockSpec((C_out, 1), lambda n: (0, 0)),
        ],
        out_specs=pl.BlockSpec((R, C_out, L), lambda n: (n, 0, 0)),
        compiler_params=pltpu.CompilerParams(
            dimension_semantics=("parallel",),
            vmem_limit_bytes=40 * 1024 * 1024,
        ),
    )(x, w2d, b2d)
